# baseline (speedup 1.0000x reference)
import functools

import jax
import jax.numpy as jnp
from jax import lax
from jax.experimental import pallas as pl
from jax.experimental.pallas import tpu as pltpu
from jax.experimental.pallas import tpu_sc as plsc

N_FIELDS = 26
VOCAB = 100000
HIDDEN = 64

NC = 2
NS = 16
L = 16
NW = NC * NS

CB = 64
ROWS = CB * N_FIELDS
NIDX = ROWS // 128
GROUPS = HIDDEN // L


def _make_kernel(B):
    bpw = B // NW
    nchunks = bpw // CB
    mesh = plsc.VectorSubcoreMesh(
        core_axis_name="c", subcore_axis_name="s",
        num_cores=NC, num_subcores=NS)

    @functools.partial(
        pl.kernel,
        out_type=jax.ShapeDtypeStruct((B, HIDDEN), jnp.float32),
        mesh=mesh,
        scratch_types=[
            pltpu.VMEM((N_FIELDS * L,), jnp.int32),
            pltpu.VMEM((ROWS,), jnp.int32),
            pltpu.VMEM((NIDX, 128), jnp.int32),
            pltpu.VMEM((ROWS, HIDDEN), jnp.float32),
            pltpu.VMEM((CB, HIDDEN), jnp.float32),
            pltpu.SemaphoreType.DMA,
        ],
        compiler_params=pltpu.CompilerParams(use_tc_tiling_on_sc=False),
    )
    def emb(x_hbm, tab_hbm, out_hbm, offs_v, xv, idxv, rows_v, outv, sem):
        wid = lax.axis_index("s") * NC + lax.axis_index("c")
        base = wid * bpw

        iota = lax.iota(jnp.int32, L)
        for j in range(N_FIELDS):
            vec = iota + (j * L)
            offs_v[pl.ds(j * L, L)] = (vec % N_FIELDS) * VOCAB

        def chunk_body(g, carry):
            row0 = base + g * CB
            pltpu.sync_copy(x_hbm.at[pl.ds(row0 * N_FIELDS, ROWS)], xv)
            for j in range(ROWS // L):
                src = xv[pl.ds(j * L, L)] + offs_v[pl.ds((j % N_FIELDS) * L, L)]
                idxv[j // 8, pl.ds((j % 8) * L, L)] = src
            copies = [
                pltpu.async_copy(
                    tab_hbm.at[idxv.at[kk]],
                    rows_v.at[pl.ds(kk * 128, 128)],
                    sem)
                for kk in range(NIDX)
            ]
            for c in copies:
                c.wait()

            def red_row(c, carry):
                rbase = c * N_FIELDS
                for h in range(GROUPS):
                    def red_field(i, acc):
                        return acc + rows_v[rbase + i, pl.ds(h * L, L)]
                    acc = lax.fori_loop(
                        0, N_FIELDS, red_field, jnp.zeros((L,), jnp.float32))
                    outv[c, pl.ds(h * L, L)] = acc / float(N_FIELDS)
                return carry
            lax.fori_loop(0, CB, red_row, 0)
            pltpu.sync_copy(outv, out_hbm.at[pl.ds(row0, CB)])
            return carry

        lax.fori_loop(0, nchunks, chunk_body, 0)

    return emb


def kernel(x, tables):
    B = x.shape[0]
    x_flat = x.astype(jnp.int32).reshape(B * N_FIELDS)
    tab_flat = tables.reshape(N_FIELDS * VOCAB, HIDDEN)
    return _make_kernel(B)(x_flat, tab_flat)


# speedup vs baseline: 1.0224x; 1.0224x over previous
import functools

import jax
import jax.numpy as jnp
from jax import lax
from jax.experimental import pallas as pl
from jax.experimental.pallas import tpu as pltpu
from jax.experimental.pallas import tpu_sc as plsc

N_FIELDS = 26
VOCAB = 100000
HIDDEN = 64

NC = 2
NS = 16
L = 16
NW = NC * NS

CB = 32
ROWS = CB * N_FIELDS
GSZ = 104
NG = ROWS // GSZ
GROUPS = HIDDEN // L
INV_N = 1.0 / N_FIELDS


def _make_kernel(B):
    bpw = B // NW
    ipw = bpw * N_FIELDS
    nchunks = bpw // CB
    mesh = plsc.VectorSubcoreMesh(
        core_axis_name="c", subcore_axis_name="s",
        num_cores=NC, num_subcores=NS)

    @functools.partial(
        pl.kernel,
        out_type=jax.ShapeDtypeStruct((B, HIDDEN), jnp.float32),
        mesh=mesh,
        scratch_types=[
            pltpu.VMEM((N_FIELDS * L,), jnp.int32),
            pltpu.VMEM((ipw,), jnp.int32),
            pltpu.VMEM((ROWS, HIDDEN), jnp.float32),
            pltpu.VMEM((ROWS, HIDDEN), jnp.float32),
            pltpu.VMEM((CB, HIDDEN), jnp.float32),
            pltpu.SemaphoreType.DMA,
            pltpu.SemaphoreType.DMA,
        ],
        compiler_params=pltpu.CompilerParams(use_tc_tiling_on_sc=False),
    )
    def emb(x_hbm, tab_hbm, out_hbm, offs_v, idxv, rows0, rows1, outv,
            sem0, sem1):
        wid = lax.axis_index("s") * NC + lax.axis_index("c")
        base = wid * bpw
        bufs = (rows0, rows1)
        sems = (sem0, sem1)

        iota = lax.iota(jnp.int32, L)
        for j in range(N_FIELDS):
            vec = iota + (j * L)
            offs_v[pl.ds(j * L, L)] = (vec % N_FIELDS) * VOCAB

        pltpu.sync_copy(x_hbm.at[pl.ds(base * N_FIELDS, ipw)], idxv)

        def idx_body(r, carry):
            p = r * (N_FIELDS * L)
            for j in range(N_FIELDS):
                sl = pl.ds(p + j * L, L)
                idxv[sl] = idxv[sl] + offs_v[pl.ds(j * L, L)]
            return carry
        lax.fori_loop(0, ipw // (N_FIELDS * L), idx_body, 0)

        def fire(k, b):
            i0 = k * ROWS
            for g in range(NG):
                pltpu.async_copy(
                    tab_hbm.at[idxv.at[pl.ds(i0 + g * GSZ, GSZ)]],
                    bufs[b].at[pl.ds(g * GSZ, GSZ)],
                    sems[b])

        def reduce_store(k, b):
            i0 = k * ROWS
            for g in range(NG):
                pltpu.make_async_copy(
                    tab_hbm.at[idxv.at[pl.ds(i0 + g * GSZ, GSZ)]],
                    bufs[b].at[pl.ds(g * GSZ, GSZ)],
                    sems[b]).wait()
            rows_v = bufs[b]

            def red_row(c, carry):
                rbase = c * N_FIELDS
                for h in range(GROUPS):
                    sl = pl.ds(h * L, L)
                    acc = rows_v[rbase, sl]
                    for i in range(1, N_FIELDS):
                        acc = acc + rows_v[rbase + i, sl]
                    outv[c, sl] = acc * INV_N
                return carry
            lax.fori_loop(0, CB, red_row, 0)
            pltpu.sync_copy(outv, out_hbm.at[pl.ds(base + k * CB, CB)])

        fire(0, 0)

        def pair_body(g, carry):
            k = 2 * g
            fire(k + 1, 1)
            reduce_store(k, 0)
            fire(k + 2, 0)
            reduce_store(k + 1, 1)
            return carry

        lax.fori_loop(0, nchunks // 2 - 1, pair_body, 0)
        fire(nchunks - 1, 1)
        reduce_store(nchunks - 2, 0)
        reduce_store(nchunks - 1, 1)

    return emb


def kernel(x, tables):
    B = x.shape[0]
    x_flat = x.astype(jnp.int32).reshape(B * N_FIELDS)
    tab_flat = tables.reshape(N_FIELDS * VOCAB, HIDDEN)
    return _make_kernel(B)(x_flat, tab_flat)
